# baseline (speedup 1.0000x reference)
import functools
import jax
import jax.numpy as jnp
from jax import lax
from jax.experimental import pallas as pl
from jax.experimental.pallas import tpu as pltpu
from jax.experimental.pallas import tpu_sc as plsc

_N_DENSE = 13
_N_SPARSE = 26
_VOCAB = 100000
_D = 16
_BATCH = 16384

_VB = 2048
_NC = 2
_NS = 16
_NW = _NC * _NS
_C = _BATCH // _NW
_GW = 128
_NG = _C // _GW
_NGRP = _C // 16


def _convert_body(tph_ref, out_ref):
  x = tph_ref[0]
  y = jnp.transpose(x, (1, 0))
  z = y.reshape(_VB // 8, 8, _D)
  out_ref[0] = jnp.concatenate([z[:, b, :] for b in range(8)], axis=1)


@functools.cache
def _make_convert_kernel():
  grid = (_N_SPARSE, _VOCAB // _VB + 1)

  return pl.pallas_call(
      _convert_body,
      grid=grid,
      in_specs=[pl.BlockSpec((1, _D, _VB), lambda f, v: (f, 0, v))],
      out_specs=pl.BlockSpec((1, _VB // 8, 128), lambda f, v: (f, v, 0)),
      out_shape=jax.ShapeDtypeStruct((_N_SPARSE, _VOCAB * _D // 128, 128),
                                     jnp.float32),
  )


@functools.cache
def _make_fm_kernel():
  mesh = plsc.VectorSubcoreMesh(
      core_axis_name="c", subcore_axis_name="s",
      num_cores=_NC, num_subcores=_NS)

  @functools.partial(
      pl.kernel,
      out_type=jax.ShapeDtypeStruct((_BATCH,), jnp.float32),
      mesh=mesh,
      scratch_types=[
          pltpu.VMEM((_N_SPARSE * _NG, _GW), jnp.int32),
          pltpu.VMEM((_N_DENSE + _N_SPARSE, _C), jnp.float32),
          pltpu.VMEM((40, 16), jnp.float32),
          pltpu.VMEM((2, _C, _D), jnp.float32),
          pltpu.VMEM((_C * _D,), jnp.float32),
          pltpu.VMEM((_C,), jnp.float32),
          pltpu.SemaphoreType.DMA,
          pltpu.SemaphoreType.DMA,
          pltpu.SemaphoreType.DMA,
      ],
      compiler_params=pltpu.CompilerParams(
          needs_layout_passes=False, use_tc_tiling_on_sc=False),
  )
  def fm_kernel(tbl_hbm, idx_hbm, xl_hbm, w_hbm, out_hbm,
                idx_v, xl_v, w_v, buf_v, acc_v, out_v,
                sem_in, sem_g0, sem_g1):
    wid = lax.axis_index("s") * _NC + lax.axis_index("c")

    cp_idx = pltpu.async_copy(idx_hbm.at[wid], idx_v, sem_in)
    cp_xl = pltpu.async_copy(xl_hbm.at[wid], xl_v, sem_in)
    cp_w = pltpu.async_copy(w_hbm, w_v, sem_in)
    cp_idx.wait()
    cp_xl.wait()
    cp_w.wait()

    sems = (sem_g0, sem_g1)

    def fire(f):
      par = f % 2
      cps = []
      for g in range(_NG):
        cps.append(pltpu.async_copy(
            tbl_hbm.at[idx_v.at[f * _NG + g]],
            buf_v.at[par, pl.ds(g * _GW, _GW)],
            sems[par]))
      return cps

    pend = fire(0)
    for f in range(_N_SPARSE):
      par = f % 2
      nxt = fire(f + 1) if f + 1 < _N_SPARSE else []
      for cp in pend:
        cp.wait()
      pend = nxt
      if f == 0:
        @plsc.parallel_loop(0, _C, unroll=8)
        def _(i):
          acc_v[pl.ds(i * _D, _D)] = buf_v[0, i]
      else:
        @plsc.parallel_loop(0, _C, unroll=8)
        def _(i):
          plsc.addupdate(acc_v.at[pl.ds(i * _D, _D)], buf_v[par, i])

    iota16 = lax.iota(jnp.int32, 16)

    for grp in range(_NGRP):
      base = grp * 16

      p = w_v[39]
      for j in range(_N_DENSE + _N_SPARSE):
        p = p + xl_v[j, pl.ds(base, 16)] * w_v[j]

      rows16 = iota16 * _D + (base * _D)
      e0 = plsc.load_gather(acc_v, [rows16])
      s = e0
      q = e0 * e0
      for d in range(1, _D):
        ed = plsc.load_gather(acc_v, [rows16 + d])
        s = s + ed
        q = q + ed * ed

      z = p + 0.5 * (s * s - q)
      out_v[pl.ds(base, 16)] = 1.0 / (1.0 + jnp.exp(-z))

    pltpu.sync_copy(out_v, out_hbm.at[pl.ds(wid * _C, _C)])

  return fm_kernel


def kernel(x, tables, W, b):
  idx = x[:, _N_DENSE:].astype(jnp.int32)
  offs = jnp.arange(_N_SPARSE, dtype=jnp.int32) * _VOCAB
  idx_off = idx + offs[None, :]
  idx_all = (idx_off.T
             .reshape(_N_SPARSE, _NW, _NG, _GW)
             .transpose(1, 0, 2, 3)
             .reshape(_NW, _N_SPARSE * _NG, _GW))
  xl = lax.reduce_precision(x, 8, 7)
  xl_all = (xl.T
            .reshape(_N_DENSE + _N_SPARSE, _NW, _C)
            .transpose(1, 0, 2))
  w = lax.reduce_precision(W[:, 0], 8, 7)
  wfull = jnp.concatenate([w, b])
  wvec = jnp.tile(wfull[:, None], (1, 16))
  tph = jnp.transpose(tables, (0, 2, 1))
  flat_tables = _make_convert_kernel()(tph).reshape(_N_SPARSE * _VOCAB, _D)

  out = _make_fm_kernel()(flat_tables, idx_all, xl_all, wvec)
  return out[:, None]


# speedup vs baseline: 1.1511x; 1.1511x over previous
import functools
import jax
import jax.numpy as jnp
from jax import lax
from jax.experimental import pallas as pl
from jax.experimental.pallas import tpu as pltpu
from jax.experimental.pallas import tpu_sc as plsc

_N_DENSE = 13
_N_SPARSE = 26
_VOCAB = 100000
_D = 16
_BATCH = 16384

_NC = 2
_NS = 16
_NW = _NC * _NS
_C = _BATCH // _NW
_GW = 128
_NG = _C // _GW
_NGRP = _C // 16


@functools.cache
def _make_fm_kernel():
  mesh = plsc.VectorSubcoreMesh(
      core_axis_name="c", subcore_axis_name="s",
      num_cores=_NC, num_subcores=_NS)

  @functools.partial(
      pl.kernel,
      out_type=jax.ShapeDtypeStruct((_BATCH,), jnp.float32),
      mesh=mesh,
      scratch_types=[
          pltpu.VMEM((_N_SPARSE * _NG, _GW), jnp.int32),
          pltpu.VMEM((_N_DENSE + _N_SPARSE, _C), jnp.float32),
          pltpu.VMEM((40, 16), jnp.float32),
          pltpu.VMEM((2, _C, _D), jnp.float32),
          pltpu.VMEM((_C * _D,), jnp.float32),
          pltpu.VMEM((_C,), jnp.float32),
          pltpu.SemaphoreType.DMA,
          pltpu.SemaphoreType.DMA,
          pltpu.SemaphoreType.DMA,
      ],
      compiler_params=pltpu.CompilerParams(
          needs_layout_passes=False, use_tc_tiling_on_sc=False),
  )
  def fm_kernel(tbl_hbm, idx_hbm, xl_hbm, w_hbm, out_hbm,
                idx_v, xl_v, w_v, buf_v, acc_v, out_v,
                sem_in, sem_g0, sem_g1):
    wid = lax.axis_index("s") * _NC + lax.axis_index("c")

    cp_idx = pltpu.async_copy(idx_hbm.at[wid], idx_v, sem_in)
    cp_xl = pltpu.async_copy(xl_hbm.at[wid], xl_v, sem_in)
    cp_w = pltpu.async_copy(w_hbm, w_v, sem_in)
    cp_idx.wait()
    cp_xl.wait()
    cp_w.wait()

    sems = (sem_g0, sem_g1)

    def fire(f):
      par = f % 2
      cps = []
      for g in range(_NG):
        cps.append(pltpu.async_copy(
            tbl_hbm.at[idx_v.at[f * _NG + g]],
            buf_v.at[par, pl.ds(g * _GW, _GW)],
            sems[par]))
      return cps

    pend = fire(0)
    for f in range(_N_SPARSE):
      par = f % 2
      nxt = fire(f + 1) if f + 1 < _N_SPARSE else []
      for cp in pend:
        cp.wait()
      pend = nxt
      if f == 0:
        @plsc.parallel_loop(0, _C, unroll=8)
        def _(i):
          acc_v[pl.ds(i * _D, _D)] = buf_v[0, i]
      else:
        @plsc.parallel_loop(0, _C, unroll=8)
        def _(i):
          plsc.addupdate(acc_v.at[pl.ds(i * _D, _D)], buf_v[par, i])

    iota16 = lax.iota(jnp.int32, 16)

    for grp in range(_NGRP):
      base = grp * 16

      p = w_v[39]
      for j in range(_N_DENSE + _N_SPARSE):
        p = p + xl_v[j, pl.ds(base, 16)] * w_v[j]

      rows16 = iota16 * _D + (base * _D)
      e0 = plsc.load_gather(acc_v, [rows16])
      s = e0
      q = e0 * e0
      for d in range(1, _D):
        ed = plsc.load_gather(acc_v, [rows16 + d])
        s = s + ed
        q = q + ed * ed

      z = p + 0.5 * (s * s - q)
      out_v[pl.ds(base, 16)] = 1.0 / (1.0 + jnp.exp(-z))

    pltpu.sync_copy(out_v, out_hbm.at[pl.ds(wid * _C, _C)])

  return fm_kernel


def kernel(x, tables, W, b):
  idx = x[:, _N_DENSE:].astype(jnp.int32)
  offs = jnp.arange(_N_SPARSE, dtype=jnp.int32) * _VOCAB
  idx_off = idx + offs[None, :]
  idx_all = (idx_off.T
             .reshape(_N_SPARSE, _NW, _NG, _GW)
             .transpose(1, 0, 2, 3)
             .reshape(_NW, _N_SPARSE * _NG, _GW))
  xl = lax.reduce_precision(x, 8, 7)
  xl_all = (xl.T
            .reshape(_N_DENSE + _N_SPARSE, _NW, _C)
            .transpose(1, 0, 2))
  w = lax.reduce_precision(W[:, 0], 8, 7)
  wfull = jnp.concatenate([w, b])
  wvec = jnp.tile(wfull[:, None], (1, 16))
  flat_tables = tables.reshape(_N_SPARSE * _VOCAB, _D)

  out = _make_fm_kernel()(flat_tables, idx_all, xl_all, wvec)
  return out[:, None]
